# baseline (speedup 1.0000x reference)
import functools

import jax
import jax.numpy as jnp
from jax.experimental import pallas as pl
from jax.experimental.pallas import tpu as pltpu

C = 16
K = 10
ALPHA = 0.9


def _lp_kernel(f_ref, ranks_ref, a_hbm, yoh_ref, m_ref, rankv_ref,
               out_ref, abuf_ref, u_ref, src_ref, sem_ref, *, bi, n, w):
    k = pl.program_id(0)
    ni = n // bi
    f = f_ref[0]
    nb = (f + bi - 1) // bi

    @pl.when(k == 0)
    def _():
        u_ref[...] = jnp.zeros_like(u_ref)

    def copy(r, slot):
        return pltpu.make_async_copy(
            a_hbm.at[pl.ds(r * bi, bi), :], abuf_ref.at[slot],
            sem_ref.at[slot])

    @pl.when((k < K) & (nb > 0))
    def _():
        copy(0, 0).start()

    def expand(i, carry):
        sl = pl.ds(i * bi, bi)
        base = ranks_ref[i * bi]
        base8 = (base // 8) * 8
        rel = rankv_ref[sl, :] - base8
        iota = jax.lax.broadcasted_iota(jnp.int32, (1, w), 1)
        e = (rel == iota).astype(jnp.float32)
        uw = u_ref[pl.ds(base8, w), :]
        ex = jnp.dot(e, uw, preferred_element_type=jnp.float32)
        m_i = m_ref[sl, :]
        val = m_i * yoh_ref[sl, :] + (1.0 - m_i) * ex
        src_ref[sl, :] = val.astype(jnp.bfloat16)

        @pl.when(k == K)
        def _():
            out_ref[sl, :] = val
        return carry

    jax.lax.fori_loop(0, ni, expand, 0)

    @pl.when(k < K)
    def _():
        def mm(r, carry):
            slot = jax.lax.rem(r, 2)

            @pl.when(r + 1 < nb)
            def _():
                copy(r + 1, 1 - slot).start()

            copy(r, slot).wait()
            z = jnp.dot(abuf_ref[slot], src_ref[...],
                        preferred_element_type=jnp.float32)
            usl = pl.ds(r * bi, bi)
            u_ref[usl, :] = jnp.clip(
                ALPHA * z + (1.0 - ALPHA) * u_ref[usl, :], 0.0, 1.0)
            return carry

        jax.lax.fori_loop(0, nb, mm, 0)


def kernel(homo_adj, y, train_mask):
    n = homo_adj.shape[0]
    bi = 400 if n % 400 == 0 else max(d for d in (8, 16, 32) if n % d == 0)
    w = bi + 8

    free = jnp.logical_not(train_mask)
    perm = jnp.argsort(jnp.where(free, 0, 1), stable=True)
    freei = free.astype(jnp.int32)
    rank = jnp.cumsum(freei) - freei
    f = jnp.sum(freei)

    a16p = homo_adj[perm].astype(jnp.bfloat16)
    y_oh = jax.nn.one_hot(y.astype(jnp.int32), C, dtype=jnp.float32)
    maskf = jnp.broadcast_to(
        train_mask.astype(jnp.float32)[:, None], (n, C))

    body = functools.partial(_lp_kernel, bi=bi, n=n, w=w)
    return pl.pallas_call(
        body,
        grid=(K + 1,),
        in_specs=[
            pl.BlockSpec(memory_space=pltpu.SMEM),
            pl.BlockSpec(memory_space=pltpu.SMEM),
            pl.BlockSpec(memory_space=pl.ANY),
            pl.BlockSpec((n, C), lambda k: (0, 0)),
            pl.BlockSpec((n, C), lambda k: (0, 0)),
            pl.BlockSpec((n, 1), lambda k: (0, 0)),
        ],
        out_specs=pl.BlockSpec((n, C), lambda k: (0, 0)),
        out_shape=jax.ShapeDtypeStruct((n, C), jnp.float32),
        scratch_shapes=[
            pltpu.VMEM((2, bi, n), jnp.bfloat16),
            pltpu.VMEM((n + 8, C), jnp.float32),
            pltpu.VMEM((n, C), jnp.bfloat16),
            pltpu.SemaphoreType.DMA((2,)),
        ],
    )(f[None], rank, a16p, y_oh, maskf, rank[:, None])
